# Initial kernel scaffold; baseline (speedup 1.0000x reference)
import jax
import jax.numpy as jnp
from jax.experimental import pallas as pl


def kernel(x, edge_index, edge_attr, batch, W1, b1, W2, b2, ee1, ee2, gamma, beta):
    raise NotImplementedError("write your pallas kernel here")


# speedup vs baseline: 1.3477x; 1.3477x over previous
import functools

import jax
import jax.numpy as jnp
from jax import lax
from jax.experimental import pallas as pl
from jax.experimental.pallas import tpu as pltpu
from jax.experimental.pallas import tpu_sc as plsc

NN = 10000
EE = 320000
DD = 128
NLAYERS = 5
NG = 64
NWORK = 32
CHSIZES = [10080] * 11 + [9840] * 4 + [9760]
BE = 128
KCH = -(-max(CHSIZES) // BE)
EPW = KCH * BE
NP = 10240
RPT = NP // 16
RBLK = 1024
NBLK = NP // RBLK
HTILES = 625


@functools.cache
def _get_edge_scatter():
    mesh = plsc.VectorSubcoreMesh(core_axis_name="c", subcore_axis_name="s")

    @functools.partial(
        pl.kernel,
        mesh=mesh,
        out_type=jax.ShapeDtypeStruct((2 * NP, DD), jnp.float32),
        scratch_types=[
            pltpu.VMEM((4, BE), jnp.int32),
            pltpu.VMEM((BE, DD), jnp.float32),
            pltpu.VMEM((BE, DD), jnp.float32),
            pltpu.VMEM_SHARED((NP, DD), jnp.float32),
        ],
    )
    def _edge_scatter(h_hbm, tbl_hbm, idx_hbm, zeros_hbm, out_hbm,
                      ibuf, rows, erows, acc):
        c = lax.axis_index("c")
        s = lax.axis_index("s")
        w = c * 16 + s
        pltpu.sync_copy(zeros_hbm, acc.at[pl.ds(s * RPT, RPT)])
        plsc.subcore_barrier()

        zeros16 = jnp.zeros((16,), jnp.float32)

        def block(b, accv):
            pltpu.sync_copy(idx_hbm.at[w * KCH + b], ibuf)
            pltpu.sync_copy(h_hbm.at[ibuf.at[0]], rows)
            pltpu.sync_copy(tbl_hbm.at[ibuf.at[1]], erows)

            def group(g, accv2):
                n16 = ibuf[3, pl.ds(g * 16, 16)]
                for ei in range(16):
                    e = g * 16 + ei
                    is_new = n16[ei] == 1
                    rowref = rows.at[e]
                    new_acc = []
                    for j in range(8):
                        m = (rows[e, pl.ds(j * 16, 16)]
                             + erows[e, pl.ds(j * 16, 16)])
                        keep = jnp.where(is_new, zeros16, accv2[j])
                        v = keep + m
                        rowref[pl.ds(j * 16, 16)] = v
                        new_acc.append(v)
                    accv2 = tuple(new_acc)
                return accv2

            accv = lax.fori_loop(0, BE // 16, group, accv)
            pltpu.sync_copy(rows, acc.at[ibuf.at[2]], add=True)
            return accv

        lax.fori_loop(0, KCH, block, tuple(zeros16 for _ in range(8)))
        plsc.subcore_barrier()
        pltpu.sync_copy(acc.at[pl.ds(s * RPT, RPT)],
                        out_hbm.at[pl.ds(c * NP + s * RPT, RPT)])

    return _edge_scatter


def _fold8(a):
    x = a[:4] + a[4:]
    x = x[:2] + x[2:]
    return x[0:1] + x[1:2]


def _mlp_body(agg_ref, h_ref, w1_ref, b1_ref, w2_ref, b2_ref, gb_ref,
              out_ref, zscr, ssum, sqsum0, sqsum1):
    p = pl.program_id(0)
    b = pl.program_id(1)

    @pl.when(p == 0)
    def _pass0():
        hin = (agg_ref[0] + agg_ref[1]) + h_ref[...]
        a1 = jnp.maximum(
            jnp.dot(hin.astype(jnp.bfloat16), w1_ref[...].astype(jnp.bfloat16),
                    preferred_element_type=jnp.float32) + b1_ref[...], 0.0)
        z = (jnp.dot(a1.astype(jnp.bfloat16), w2_ref[...].astype(jnp.bfloat16),
                     preferred_element_type=jnp.float32) + b2_ref[...])
        zscr[pl.ds(b * RBLK, RBLK), :] = z

        @pl.when(b == 0)
        def _():
            ssum[...] = jnp.zeros_like(ssum)

        def tile(t, carry):
            gt = b * 128 + t

            @pl.when(gt < 2 * HTILES)
            def _():
                ssum[...] += zscr[pl.ds(b * RBLK + t * 8, 8), :]

            return carry
        lax.fori_loop(0, 128, tile, 0)

    @pl.when(p == 1)
    def _pass1():
        mu = _fold8(ssum[...]) * (1.0 / NN)

        @pl.when(b == 0)
        def _():
            sqsum0[...] = jnp.zeros_like(sqsum0)
            sqsum1[...] = jnp.zeros_like(sqsum1)

        def tile(t, carry):
            gt = b * 128 + t
            d = zscr[pl.ds(b * RBLK + t * 8, 8), :] - mu
            blk = d * d

            @pl.when(gt < HTILES)
            def _():
                sqsum0[...] += blk

            @pl.when(jnp.logical_and(gt >= HTILES, gt < 2 * HTILES))
            def _():
                sqsum1[...] += blk

            return carry
        lax.fori_loop(0, 128, tile, 0)

    @pl.when(p == 2)
    def _pass2():
        mu = _fold8(ssum[...]) * (1.0 / NN)
        var = (_fold8(sqsum0[...]) + _fold8(sqsum1[...])) * (1.0 / NN)
        z = zscr[pl.ds(b * RBLK, RBLK), :]
        zn = gb_ref[0:1, :] * (z - mu) / jnp.sqrt(var + 1e-5) + gb_ref[1:2, :]
        out_ref[...] = jnp.maximum(zn, 0.0)


def _mlp_layer(agg2, h, w1, b1r, w2, b2r, gb):
    return pl.pallas_call(
        _mlp_body,
        grid=(3, NBLK),
        in_specs=[
            pl.BlockSpec((2, RBLK, DD), lambda p, b: (0, b, 0)),
            pl.BlockSpec((RBLK, DD), lambda p, b: (b, 0)),
            pl.BlockSpec((DD, 2 * DD), lambda p, b: (0, 0)),
            pl.BlockSpec((1, 2 * DD), lambda p, b: (0, 0)),
            pl.BlockSpec((2 * DD, DD), lambda p, b: (0, 0)),
            pl.BlockSpec((1, DD), lambda p, b: (0, 0)),
            pl.BlockSpec((2, DD), lambda p, b: (0, 0)),
        ],
        out_specs=pl.BlockSpec((RBLK, DD), lambda p, b: (b, 0)),
        out_shape=jax.ShapeDtypeStruct((NP, DD), jnp.float32),
        scratch_shapes=[pltpu.VMEM((NP, DD), jnp.float32),
                        pltpu.VMEM((8, DD), jnp.float32),
                        pltpu.VMEM((8, DD), jnp.float32),
                        pltpu.VMEM((8, DD), jnp.float32)],
    )(agg2, h, w1, b1r, w2, b2r, gb)


def _pool_body(h_ref, b_ref, out_ref, acc, cnt):
    b = pl.program_id(0)

    @pl.when(b == 0)
    def _():
        acc[...] = jnp.zeros_like(acc)
        cnt[...] = jnp.zeros_like(cnt)

    ids = jax.lax.broadcasted_iota(jnp.int32, (NG, RBLK), 0)
    oh = (ids == b_ref[...]).astype(jnp.float32)
    acc[...] += jax.lax.dot_general(oh, h_ref[...], (((1,), (0,)), ((), ())),
                                    preferred_element_type=jnp.float32,
                                    precision=jax.lax.Precision.HIGHEST)
    cnt[...] += jnp.sum(oh, axis=1, keepdims=True)

    @pl.when(b == NBLK - 1)
    def _():
        out_ref[...] = acc[...] / jnp.maximum(cnt[...], 1.0)


def _pool(h, batchp):
    return pl.pallas_call(
        _pool_body,
        grid=(NBLK,),
        in_specs=[
            pl.BlockSpec((RBLK, DD), lambda b: (b, 0)),
            pl.BlockSpec((1, RBLK), lambda b: (0, b)),
        ],
        out_specs=pl.BlockSpec((NG, DD), lambda b: (0, 0)),
        out_shape=jax.ShapeDtypeStruct((NG, DD), jnp.float32),
        scratch_shapes=[pltpu.VMEM((NG, DD), jnp.float32),
                        pltpu.VMEM((NG, DD), jnp.float32)],
    )(h, batchp)


def kernel(x, edge_index, edge_attr, batch, W1, b1, W2, b2, ee1, ee2,
           gamma, beta):
    src = edge_index[0]
    dst = edge_index[1]
    q = edge_attr[:, 0] * 3 + edge_attr[:, 1]

    order = jnp.argsort(dst, stable=True)
    srcs = src[order]
    dsts = dst[order]
    qs = q[order]

    import numpy as _np
    sizes = _np.array(CHSIZES * 2, _np.int32)
    bpos = _np.concatenate([[0], _np.cumsum(sizes)[:-1]]).astype(_np.int32)
    is_bound = _np.zeros((EE,), bool)
    is_bound[bpos] = True
    bnd = jnp.asarray(is_bound)

    isnew = jnp.logical_or(
        bnd, jnp.concatenate([jnp.ones((1,), jnp.bool_),
                              dsts[1:] != dsts[:-1]])).astype(jnp.int32)
    islast = jnp.logical_or(
        jnp.concatenate([bnd[1:], jnp.ones((1,), jnp.bool_)]),
        jnp.concatenate([dsts[:-1] != dsts[1:], jnp.ones((1,), jnp.bool_)]))
    dlast = jnp.where(islast, dsts, NN)

    rel = jnp.arange(EPW, dtype=jnp.int32)[None, :]
    pos = jnp.minimum(jnp.asarray(bpos)[:, None] + rel, EE - 1)
    valid = rel < jnp.asarray(sizes)[:, None]
    def winval(a, padval):
        return jnp.where(valid, jnp.take(a, pos, axis=0), padval
                         ).reshape(NWORK, KCH, BE)
    idxw = jnp.stack([winval(srcs, 0), winval(qs, 18),
                      winval(dlast, NN), winval(isnew, 1)], axis=2
                     ).reshape(NWORK * KCH, 4, BE)

    qi = jnp.arange(32, dtype=jnp.int32)
    a_i = jnp.clip(qi // 3, 0, 5)
    b_i = qi % 3
    tbl2 = jnp.where((qi < 18)[None, :, None],
                     ee1[:, a_i, :] + ee2[:, b_i, :], 0.0)

    zrows = jnp.zeros((RPT, DD), jnp.float32)
    h = jnp.concatenate([x, jnp.zeros((NP - NN, DD), jnp.float32)])
    b1r = b1.reshape(NLAYERS, 1, 2 * DD)
    b2r = b2.reshape(NLAYERS, 1, DD)
    gb = jnp.stack([gamma, beta], axis=1)

    es = _get_edge_scatter()
    for i in range(NLAYERS):
        agg2 = es(h, tbl2[i], idxw, zrows).reshape(2, NP, DD)
        h = _mlp_layer(agg2, h, W1[i], b1r[i], W2[i], b2r[i], gb[i])

    batchp = jnp.concatenate([batch, jnp.full((NP - NN,), NG, jnp.int32)]
                             ).reshape(1, NP)
    gemb = _pool(h, batchp)
    return (h[:NN], gemb)
